# Initial kernel scaffold; baseline (speedup 1.0000x reference)
import jax
import jax.numpy as jnp
from jax.experimental import pallas as pl


def kernel(x, skip, params):
    raise NotImplementedError("write your pallas kernel here")


# speedup vs baseline: 26.9961x; 26.9961x over previous
import functools

import jax
import jax.numpy as jnp
from jax.experimental import pallas as pl
from jax.experimental.pallas import tpu as pltpu

F32 = jnp.float32
_INTERPRET = False


def _cparams(grid_len):
    return pltpu.CompilerParams(
        dimension_semantics=("parallel",) * grid_len,
        vmem_limit_bytes=56 * 1024 * 1024,
    )


def _sigmoid(x):
    return 1.0 / (1.0 + jnp.exp(-x))


def _silu(x):
    return x * _sigmoid(x)


def _gelu(x):
    y = 0.7978845608028654 * (x + 0.044715 * x * x * x)
    return x * _sigmoid(2.0 * y)


def _softplus(x):
    return jnp.maximum(x, 0.0) + jnp.log(1.0 + jnp.exp(-jnp.abs(x)))


def _ln(x2, g, b):
    mu = jnp.mean(x2, axis=-1, keepdims=True)
    xc = x2 - mu
    var = jnp.mean(xc * xc, axis=-1, keepdims=True)
    return xc * jax.lax.rsqrt(var + 1e-5) * g + b


def _gn(x2, gmat, gmat_t, g, b):
    n = x2.shape[0] * (x2.shape[1] // gmat.shape[1])
    s1 = jnp.dot(jnp.sum(x2, axis=0, keepdims=True), gmat) / n
    s2 = jnp.dot(jnp.sum(x2 * x2, axis=0, keepdims=True), gmat) / n
    var = s2 - s1 * s1
    rs = jax.lax.rsqrt(var + 1e-5)
    muf = jnp.dot(s1, gmat_t)
    rsf = jnp.dot(rs, gmat_t)
    return (x2 - muf) * rsf * g + b


def _shift2d(x3, a, b):
    H, W, C = x3.shape
    dt = x3.dtype
    if a > 0:
        x3 = jnp.concatenate([x3[a:], jnp.zeros((a, W, C), dt)], axis=0)
    elif a < 0:
        x3 = jnp.concatenate([jnp.zeros((-a, W, C), dt), x3[:a]], axis=0)
    if b > 0:
        x3 = jnp.concatenate([x3[:, b:], jnp.zeros((H, b, C), dt)], axis=1)
    elif b < 0:
        x3 = jnp.concatenate([jnp.zeros((H, -b, C), dt), x3[:, :b]], axis=1)
    return x3


def _conv3(x3, w, bias):
    H, W, _ = x3.shape
    acc = jnp.broadcast_to(bias, (H * W, w.shape[3])).astype(F32)
    for dy in range(3):
        for dx in range(3):
            xs = _shift2d(x3, dy - 1, dx - 1).reshape(H * W, x3.shape[2])
            acc = acc + jnp.dot(xs, w[dy, dx], preferred_element_type=F32)
    return acc


def _dwconv3(x3, w, bias):
    H, W, C = x3.shape
    acc = jnp.broadcast_to(bias, (H * W, C)).astype(F32)
    for dy in range(3):
        for dx in range(3):
            xs = _shift2d(x3, dy - 1, dx - 1).reshape(H * W, C)
            acc = acc + xs * w[dy, dx][None, :]
    return acc


def _front_kernel(xu_ref, sk_ref, upw, upb, g1g, g1b, c1w, c1b, g2g, g2b,
                  c2w, c2b, scg, scb, scw, scb2, gma, gma_t, gmb, gmb_t,
                  out_ref):
    H, W, Ci = xu_ref.shape[1], xu_ref.shape[2], xu_ref.shape[3]
    L = H * W
    xu = xu_ref[0]
    xuc = _conv3(xu, upw, upb[...])
    hcat = jnp.concatenate([xuc, sk_ref[0].reshape(L, -1)], axis=-1)
    t = _silu(_gn(hcat, gma[...], gma_t[...], g1g[...], g1b[...]))
    h1 = _conv3(t.reshape(H, W, Ci), c1w, c1b[...])
    t2 = _silu(_gn(h1, gmb[...], gmb_t[...], g2g[...], g2b[...]))
    h2 = _conv3(t2.reshape(H, W, h1.shape[1]), c2w, c2b[...])
    s = _silu(_gn(hcat, gma[...], gma_t[...], scg[...], scb[...]))
    s = jnp.dot(s, scw[...], preferred_element_type=F32) + scb2[...]
    out_ref[0] = s + h2


def _f1_kernel(h_ref, lng, lnb, ipw, dww, dwb, xc_ref, z_ref, sc_x1):
    L, d = h_ref.shape[1], h_ref.shape[2]
    H = W = 64
    Di = dwb.shape[1]
    nch = 4
    rows = L // nch
    for c in range(nch):
        r0 = c * rows
        hx = h_ref[0, r0:r0 + rows, :]
        ln = _ln(hx, lng[...], lnb[...])
        xz = jnp.dot(ln, ipw[...], preferred_element_type=F32)
        z_ref[0, r0:r0 + rows, :] = xz[:, Di:]
        sc_x1[(r0 // W):(r0 + rows) // W] = xz[:, :Di].reshape(rows // W, W, Di)
    xc = _dwconv3(sc_x1[...], dww, dwb[...])
    xc_ref[0] = _silu(xc)


def _ssm_kernel(xcr_ref, xct_ref, xpw, dtw, dtb, A_ref, ysl_ref, yst_ref,
                sc_dts, sc_u, sc_b, sc_c):
    L, Di = xcr_ref.shape[1], xcr_ref.shape[2]
    R = 8
    N = 16
    nch = 4
    rows = L // nch
    ngrp = L // 8
    for k in range(4):
        xsel_ref = xcr_ref if k % 2 == 0 else xct_ref
        for c in range(nch):
            r0 = c * rows
            xs_c = xsel_ref[0, r0:r0 + rows, :]
            xd = jnp.dot(xs_c, xpw[k], preferred_element_type=F32)
            dt = _softplus(
                jnp.dot(xd[:, 0:R], dtw[k], preferred_element_type=F32)
                + dtb[k])
            sc_dts[r0:r0 + rows, :] = dt
            sc_u[r0:r0 + rows, :] = dt * xs_c
            sc_b[r0:r0 + rows, :] = xd[:, R:R + N]
            sc_c[r0:r0 + rows, :] = xd[:, R + N:R + 2 * N]
        A_k = A_ref[k]
        rev = k >= 2
        tgt = ysl_ref if k % 2 == 0 else yst_ref

        def group(i, h, *, _rev=rev, _acc=rev, _tgt=tgt, _A=A_k):
            if _rev:
                base = pl.multiple_of((ngrp - 1 - i) * 8, 8)
            else:
                base = pl.multiple_of(i * 8, 8)
            dt8 = sc_dts[pl.ds(base, 8), :]
            u8 = sc_u[pl.ds(base, 8), :]
            b8 = sc_b[pl.ds(base, 8), :]
            c8 = sc_c[pl.ds(base, 8), :]
            dA8 = jnp.exp(dt8[:, None, :] * _A[None, :, :])
            bu8 = u8[:, None, :] * b8[:, :, None]
            cb8 = c8[:, :, None]
            out_rows = [None] * 8
            order = range(7, -1, -1) if _rev else range(8)
            for q in order:
                h = dA8[q] * h + bu8[q]
                out_rows[q] = jnp.sum(h * cb8[q], axis=0, keepdims=True)
            r8 = jnp.concatenate(out_rows, axis=0)
            if _acc:
                _tgt[0, pl.ds(base, 8), :] = _tgt[0, pl.ds(base, 8), :] + r8
            else:
                _tgt[0, pl.ds(base, 8), :] = r8
            return h

        h0 = jnp.zeros((N, Di), F32)
        jax.lax.fori_loop(0, ngrp, group, h0)


def _post_kernel(ysl_ref, ystu_ref, xcr_ref, z_ref, hin_ref, ong, onb, dsum,
                 opw, l2g, l2b, w1, b1, w2, b2, out_ref):
    L = ysl_ref.shape[1]
    nch = 4
    rows = L // nch
    for c in range(nch):
        r0 = c * rows
        sl = slice(r0, r0 + rows)
        y = ysl_ref[0, sl, :] + ystu_ref[0, sl, :] + dsum[...] * xcr_ref[0, sl, :]
        y = _ln(y, ong[...], onb[...])
        y = y * _silu(z_ref[0, sl, :])
        o = jnp.dot(y, opw[...], preferred_element_type=F32)
        hh = hin_ref[0, sl, :] + o
        m = _ln(hh, l2g[...], l2b[...])
        m = _gelu(jnp.dot(m, w1[...], preferred_element_type=F32) + b1[...])
        m = jnp.dot(m, w2[...], preferred_element_type=F32) + b2[...]
        out_ref[0, sl, :] = hh + m


def _res_kernel(h_ref, g1g, g1b, c1w, c1b, g2g, g2b, c2w, c2b, gm, gm_t,
                out_ref):
    H = W = 64
    L, d = h_ref.shape[1], h_ref.shape[2]
    hx = h_ref[0]
    t = _silu(_gn(hx, gm[...], gm_t[...], g1g[...], g1b[...]))
    h1 = _conv3(t.reshape(H, W, d), c1w, c1b[...])
    t2 = _silu(_gn(h1, gm[...], gm_t[...], g2g[...], g2b[...]))
    h2 = _conv3(t2.reshape(H, W, d), c2w, c2b[...])
    out_ref[0] = hx + h2


def _gmat(C, G):
    return (jnp.arange(C)[:, None] // (C // G) ==
            jnp.arange(G)[None, :]).astype(F32)


def _row(v):
    return v.reshape(1, -1).astype(F32)


def _cw(w):
    return w.transpose(2, 3, 1, 0).astype(F32)


def _spec_b(shape):
    n = len(shape)
    return pl.BlockSpec((1,) + shape[1:],
                        lambda b, *, _n=n: (b,) + (0,) * (_n - 1))


def _spec_w(shape):
    n = len(shape)
    return pl.BlockSpec(shape, lambda b, *, _n=n: (0,) * _n)


def _call(kfn, name, B, inputs_b, inputs_w, out_shapes, scratch=()):
    in_specs = ([_spec_b(a.shape) for a in inputs_b] +
                [_spec_w(a.shape) for a in inputs_w])
    out_specs = [_spec_b(s.shape) for s in out_shapes]
    single = len(out_shapes) == 1
    r = pl.pallas_call(
        kfn,
        grid=(B,),
        in_specs=in_specs,
        out_specs=out_specs[0] if single else out_specs,
        out_shape=out_shapes[0] if single else list(out_shapes),
        scratch_shapes=list(scratch),
        compiler_params=_cparams(1),
        name=name,
        interpret=_INTERPRET,
    )(*inputs_b, *inputs_w)
    return r


def _tr_hw(a, H, W):
    B, L, C = a.shape
    return a.reshape(B, H, W, C).transpose(0, 2, 1, 3).reshape(B, L, C)


def kernel(x, skip, params):
    B, Cin, Hs, Ws = x.shape
    d = skip.shape[1]
    H, W = skip.shape[2], skip.shape[3]
    L = H * W

    xu = jnp.repeat(jnp.repeat(x, 2, axis=2), 2, axis=3)
    xu_n = xu.transpose(0, 2, 3, 1).astype(F32)
    sk_n = skip.transpose(0, 2, 3, 1).astype(F32)

    pr = params['in_res']
    gma, gmb = _gmat(Cin, 8), _gmat(d, 8)
    front_w = [
        _cw(params['up_conv_w']), _row(params['up_conv_b']),
        _row(pr['gn1_g']), _row(pr['gn1_b']), _cw(pr['conv1_w']),
        _row(pr['conv1_b']), _row(pr['gn2_g']), _row(pr['gn2_b']),
        _cw(pr['conv2_w']), _row(pr['conv2_b']), _row(pr['sc_gn_g']),
        _row(pr['sc_gn_b']), pr['sc_w'][:, :, 0, 0].T.astype(F32),
        _row(pr['sc_b']), gma, gma.T, gmb, gmb.T,
    ]
    h = _call(_front_kernel, "front", B, [xu_n, sk_n], front_w,
              [jax.ShapeDtypeStruct((B, L, d), F32)])

    for layer in params['layers']:
        for vp in (layer['vss1'], layer['vss2']):
            Di = vp['conv_w'].shape[0]
            f1_w = [
                _row(vp['ln1_g']), _row(vp['ln1_b']),
                vp['in_proj_w'].T.astype(F32),
                vp['conv_w'].transpose(2, 3, 1, 0)[:, :, 0, :].astype(F32),
                _row(vp['conv_b']),
            ]
            xc, z = _call(
                _f1_kernel, "vss_f1", B, [h], f1_w,
                [jax.ShapeDtypeStruct((B, L, Di), F32),
                 jax.ShapeDtypeStruct((B, L, Di), F32)],
                scratch=[pltpu.VMEM((H, W, Di), F32)])
            xct = _tr_hw(xc, H, W)
            ssm_w = [
                vp['x_proj_w'].transpose(0, 2, 1).astype(F32),
                vp['dt_w'].transpose(0, 2, 1).astype(F32),
                vp['dt_b'][:, None, :].astype(F32),
                (-jnp.exp(vp['A_logs'])).transpose(0, 2, 1).astype(F32),
            ]
            ysl, yst = _call(
                _ssm_kernel, "ssm", B, [xc, xct], ssm_w,
                [jax.ShapeDtypeStruct((B, L, Di), F32),
                 jax.ShapeDtypeStruct((B, L, Di), F32)],
                scratch=[pltpu.VMEM((L, Di), F32), pltpu.VMEM((L, Di), F32),
                         pltpu.VMEM((L, 16), F32), pltpu.VMEM((L, 16), F32)])
            ystu = _tr_hw(yst, H, W)
            post_w = [
                _row(vp['out_norm_g']), _row(vp['out_norm_b']),
                _row(vp['Ds'].sum(axis=0)),
                vp['out_proj_w'].T.astype(F32),
                _row(vp['ln2_g']), _row(vp['ln2_b']),
                vp['mlp_w1'].T.astype(F32), _row(vp['mlp_b1']),
                vp['mlp_w2'].T.astype(F32), _row(vp['mlp_b2']),
            ]
            h = _call(_post_kernel, "vss_post", B, [ysl, ystu, xc, z, h],
                      post_w, [jax.ShapeDtypeStruct((B, L, d), F32)])
        rp = layer['res']
        res_w = [
            _row(rp['gn1_g']), _row(rp['gn1_b']), _cw(rp['conv1_w']),
            _row(rp['conv1_b']), _row(rp['gn2_g']), _row(rp['gn2_b']),
            _cw(rp['conv2_w']), _row(rp['conv2_b']), gmb, gmb.T,
        ]
        h = _call(_res_kernel, "res", B, [h], res_w,
                  [jax.ShapeDtypeStruct((B, L, d), F32)])

    return h.reshape(B, H, W, d).transpose(0, 3, 1, 2)


# speedup vs baseline: 35.8227x; 1.3270x over previous
import functools

import jax
import jax.numpy as jnp
from jax.experimental import pallas as pl
from jax.experimental.pallas import tpu as pltpu

F32 = jnp.float32
_INTERPRET = False


def _cparams(grid_len):
    return pltpu.CompilerParams(
        dimension_semantics=("parallel",) * grid_len,
        vmem_limit_bytes=56 * 1024 * 1024,
    )


def _sigmoid(x):
    return 1.0 / (1.0 + jnp.exp(-x))


def _silu(x):
    return x * _sigmoid(x)


def _gelu(x):
    y = 0.7978845608028654 * (x + 0.044715 * x * x * x)
    return x * _sigmoid(2.0 * y)


def _softplus(x):
    return jnp.maximum(x, 0.0) + jnp.log(1.0 + jnp.exp(-jnp.abs(x)))


def _ln(x2, g, b):
    mu = jnp.mean(x2, axis=-1, keepdims=True)
    xc = x2 - mu
    var = jnp.mean(xc * xc, axis=-1, keepdims=True)
    return xc * jax.lax.rsqrt(var + 1e-5) * g + b


def _gn(x2, gmat, gmat_t, g, b):
    n = x2.shape[0] * (x2.shape[1] // gmat.shape[1])
    s1 = jnp.dot(jnp.sum(x2, axis=0, keepdims=True), gmat) / n
    s2 = jnp.dot(jnp.sum(x2 * x2, axis=0, keepdims=True), gmat) / n
    var = s2 - s1 * s1
    rs = jax.lax.rsqrt(var + 1e-5)
    muf = jnp.dot(s1, gmat_t)
    rsf = jnp.dot(rs, gmat_t)
    return (x2 - muf) * rsf * g + b


def _shift2d(x3, a, b):
    H, W, C = x3.shape
    dt = x3.dtype
    if a > 0:
        x3 = jnp.concatenate([x3[a:], jnp.zeros((a, W, C), dt)], axis=0)
    elif a < 0:
        x3 = jnp.concatenate([jnp.zeros((-a, W, C), dt), x3[:a]], axis=0)
    if b > 0:
        x3 = jnp.concatenate([x3[:, b:], jnp.zeros((H, b, C), dt)], axis=1)
    elif b < 0:
        x3 = jnp.concatenate([jnp.zeros((H, -b, C), dt), x3[:, :b]], axis=1)
    return x3


def _conv3(x3, w, bias):
    H, W, _ = x3.shape
    acc = jnp.broadcast_to(bias, (H * W, w.shape[3])).astype(F32)
    for dy in range(3):
        for dx in range(3):
            xs = _shift2d(x3, dy - 1, dx - 1).reshape(H * W, x3.shape[2])
            acc = acc + jnp.dot(xs, w[dy, dx], preferred_element_type=F32)
    return acc


def _dwconv3(x3, w, bias):
    H, W, C = x3.shape
    acc = jnp.broadcast_to(bias, (H * W, C)).astype(F32)
    for dy in range(3):
        for dx in range(3):
            xs = _shift2d(x3, dy - 1, dx - 1).reshape(H * W, C)
            acc = acc + xs * w[dy, dx][None, :]
    return acc


def _front_kernel(xu_ref, sk_ref, upw, upb, g1g, g1b, c1w, c1b, g2g, g2b,
                  c2w, c2b, scg, scb, scw, scb2, gma, gma_t, gmb, gmb_t,
                  out_ref):
    H, W, Ci = xu_ref.shape[1], xu_ref.shape[2], xu_ref.shape[3]
    L = H * W
    xu = xu_ref[0]
    xuc = _conv3(xu, upw, upb[...])
    hcat = jnp.concatenate([xuc, sk_ref[0].reshape(L, -1)], axis=-1)
    t = _silu(_gn(hcat, gma[...], gma_t[...], g1g[...], g1b[...]))
    h1 = _conv3(t.reshape(H, W, Ci), c1w, c1b[...])
    t2 = _silu(_gn(h1, gmb[...], gmb_t[...], g2g[...], g2b[...]))
    h2 = _conv3(t2.reshape(H, W, h1.shape[1]), c2w, c2b[...])
    s = _silu(_gn(hcat, gma[...], gma_t[...], scg[...], scb[...]))
    s = jnp.dot(s, scw[...], preferred_element_type=F32) + scb2[...]
    out_ref[0] = s + h2


def _f1_kernel(h_ref, lng, lnb, ipw, dww, dwb, xc_ref, z_ref, sc_x1):
    L, d = h_ref.shape[1], h_ref.shape[2]
    H = W = 64
    Di = dwb.shape[1]
    nch = 4
    rows = L // nch
    for c in range(nch):
        r0 = c * rows
        hx = h_ref[0, r0:r0 + rows, :]
        ln = _ln(hx, lng[...], lnb[...])
        xz = jnp.dot(ln, ipw[...], preferred_element_type=F32)
        z_ref[0, r0:r0 + rows, :] = xz[:, Di:]
        sc_x1[(r0 // W):(r0 + rows) // W] = xz[:, :Di].reshape(rows // W, W, Di)
    xc = _dwconv3(sc_x1[...], dww, dwb[...])
    xc_ref[0] = _silu(xc)


def _ssm_kernel(xcr_ref, xct_ref, xpw, dtw, dtb, A_ref, ysl_ref, yst_ref,
                sc_dts, sc_b, sc_c):
    L, Di = xcr_ref.shape[1], xcr_ref.shape[2]
    R = 8
    N = 16
    nch = 4
    rows = L // nch
    ngrp = L // 8
    for pair in range(2):
        for kk in range(2):
            k = pair * 2 + kk
            xsel_ref = xcr_ref if kk == 0 else xct_ref
            for c in range(nch):
                r0 = c * rows
                xs_c = xsel_ref[0, r0:r0 + rows, :]
                xd = jnp.dot(xs_c, xpw[k], preferred_element_type=F32)
                dt = _softplus(
                    jnp.dot(xd[:, 0:R], dtw[k], preferred_element_type=F32)
                    + dtb[k])
                sc_dts[kk, r0:r0 + rows, :] = dt
                sc_b[kk, r0:r0 + rows, :] = xd[:, R:R + N]
                sc_c[kk, r0:r0 + rows, :] = xd[:, R + N:R + 2 * N]
        rev = pair == 1
        A0 = A_ref[pair * 2]
        A1 = A_ref[pair * 2 + 1]

        def group(i, hs, *, _rev=rev, _A0=A0, _A1=A1):
            ha, hb = hs
            if _rev:
                base = pl.multiple_of((ngrp - 1 - i) * 8, 8)
            else:
                base = pl.multiple_of(i * 8, 8)
            dsl = pl.ds(base, 8)
            dt8a = sc_dts[0, dsl, :]
            dt8b = sc_dts[1, dsl, :]
            u8a = dt8a * xcr_ref[0, dsl, :]
            u8b = dt8b * xct_ref[0, dsl, :]
            b8a = sc_b[0, dsl, :]
            b8b = sc_b[1, dsl, :]
            c8a = sc_c[0, dsl, :]
            c8b = sc_c[1, dsl, :]
            dA8a = jnp.exp(dt8a[:, None, :] * _A0[None, :, :])
            dA8b = jnp.exp(dt8b[:, None, :] * _A1[None, :, :])
            bu8a = u8a[:, None, :] * b8a[:, :, None]
            bu8b = u8b[:, None, :] * b8b[:, :, None]
            cb8a = c8a[:, :, None]
            cb8b = c8b[:, :, None]
            rows_a = [None] * 8
            rows_b = [None] * 8
            order = range(7, -1, -1) if _rev else range(8)
            for q in order:
                ha = dA8a[q] * ha + bu8a[q]
                hb = dA8b[q] * hb + bu8b[q]
                rows_a[q] = jnp.sum(ha * cb8a[q], axis=0, keepdims=True)
                rows_b[q] = jnp.sum(hb * cb8b[q], axis=0, keepdims=True)
            r8a = jnp.concatenate(rows_a, axis=0)
            r8b = jnp.concatenate(rows_b, axis=0)
            if _rev:
                ysl_ref[0, dsl, :] = ysl_ref[0, dsl, :] + r8a
                yst_ref[0, dsl, :] = yst_ref[0, dsl, :] + r8b
            else:
                ysl_ref[0, dsl, :] = r8a
                yst_ref[0, dsl, :] = r8b
            return ha, hb

        h0 = (jnp.zeros((N, Di), F32), jnp.zeros((N, Di), F32))
        jax.lax.fori_loop(0, ngrp, group, h0)


def _post_kernel(ysl_ref, ystu_ref, xcr_ref, z_ref, hin_ref, ong, onb, dsum,
                 opw, l2g, l2b, w1, b1, w2, b2, out_ref):
    L = ysl_ref.shape[1]
    nch = 4
    rows = L // nch
    for c in range(nch):
        r0 = c * rows
        sl = slice(r0, r0 + rows)
        y = ysl_ref[0, sl, :] + ystu_ref[0, sl, :] + dsum[...] * xcr_ref[0, sl, :]
        y = _ln(y, ong[...], onb[...])
        y = y * _silu(z_ref[0, sl, :])
        o = jnp.dot(y, opw[...], preferred_element_type=F32)
        hh = hin_ref[0, sl, :] + o
        m = _ln(hh, l2g[...], l2b[...])
        m = _gelu(jnp.dot(m, w1[...], preferred_element_type=F32) + b1[...])
        m = jnp.dot(m, w2[...], preferred_element_type=F32) + b2[...]
        out_ref[0, sl, :] = hh + m


def _res_kernel(h_ref, g1g, g1b, c1w, c1b, g2g, g2b, c2w, c2b, gm, gm_t,
                out_ref):
    H = W = 64
    L, d = h_ref.shape[1], h_ref.shape[2]
    hx = h_ref[0]
    t = _silu(_gn(hx, gm[...], gm_t[...], g1g[...], g1b[...]))
    h1 = _conv3(t.reshape(H, W, d), c1w, c1b[...])
    t2 = _silu(_gn(h1, gm[...], gm_t[...], g2g[...], g2b[...]))
    h2 = _conv3(t2.reshape(H, W, d), c2w, c2b[...])
    out_ref[0] = hx + h2


def _gmat(C, G):
    return (jnp.arange(C)[:, None] // (C // G) ==
            jnp.arange(G)[None, :]).astype(F32)


def _row(v):
    return v.reshape(1, -1).astype(F32)


def _cw(w):
    return w.transpose(2, 3, 1, 0).astype(F32)


def _spec_b(shape):
    n = len(shape)
    return pl.BlockSpec((1,) + shape[1:],
                        lambda b, *, _n=n: (b,) + (0,) * (_n - 1))


def _spec_w(shape):
    n = len(shape)
    return pl.BlockSpec(shape, lambda b, *, _n=n: (0,) * _n)


def _call(kfn, name, B, inputs_b, inputs_w, out_shapes, scratch=()):
    in_specs = ([_spec_b(a.shape) for a in inputs_b] +
                [_spec_w(a.shape) for a in inputs_w])
    out_specs = [_spec_b(s.shape) for s in out_shapes]
    single = len(out_shapes) == 1
    r = pl.pallas_call(
        kfn,
        grid=(B,),
        in_specs=in_specs,
        out_specs=out_specs[0] if single else out_specs,
        out_shape=out_shapes[0] if single else list(out_shapes),
        scratch_shapes=list(scratch),
        compiler_params=_cparams(1),
        name=name,
        interpret=_INTERPRET,
    )(*inputs_b, *inputs_w)
    return r


def _tr_hw(a, H, W):
    B, L, C = a.shape
    return a.reshape(B, H, W, C).transpose(0, 2, 1, 3).reshape(B, L, C)


def kernel(x, skip, params):
    B, Cin, Hs, Ws = x.shape
    d = skip.shape[1]
    H, W = skip.shape[2], skip.shape[3]
    L = H * W

    xu = jnp.repeat(jnp.repeat(x, 2, axis=2), 2, axis=3)
    xu_n = xu.transpose(0, 2, 3, 1).astype(F32)
    sk_n = skip.transpose(0, 2, 3, 1).astype(F32)

    pr = params['in_res']
    gma, gmb = _gmat(Cin, 8), _gmat(d, 8)
    front_w = [
        _cw(params['up_conv_w']), _row(params['up_conv_b']),
        _row(pr['gn1_g']), _row(pr['gn1_b']), _cw(pr['conv1_w']),
        _row(pr['conv1_b']), _row(pr['gn2_g']), _row(pr['gn2_b']),
        _cw(pr['conv2_w']), _row(pr['conv2_b']), _row(pr['sc_gn_g']),
        _row(pr['sc_gn_b']), pr['sc_w'][:, :, 0, 0].T.astype(F32),
        _row(pr['sc_b']), gma, gma.T, gmb, gmb.T,
    ]
    h = _call(_front_kernel, "front", B, [xu_n, sk_n], front_w,
              [jax.ShapeDtypeStruct((B, L, d), F32)])

    for layer in params['layers']:
        for vp in (layer['vss1'], layer['vss2']):
            Di = vp['conv_w'].shape[0]
            f1_w = [
                _row(vp['ln1_g']), _row(vp['ln1_b']),
                vp['in_proj_w'].T.astype(F32),
                vp['conv_w'].transpose(2, 3, 1, 0)[:, :, 0, :].astype(F32),
                _row(vp['conv_b']),
            ]
            xc, z = _call(
                _f1_kernel, "vss_f1", B, [h], f1_w,
                [jax.ShapeDtypeStruct((B, L, Di), F32),
                 jax.ShapeDtypeStruct((B, L, Di), F32)],
                scratch=[pltpu.VMEM((H, W, Di), F32)])
            xct = _tr_hw(xc, H, W)
            ssm_w = [
                vp['x_proj_w'].transpose(0, 2, 1).astype(F32),
                vp['dt_w'].transpose(0, 2, 1).astype(F32),
                vp['dt_b'][:, None, :].astype(F32),
                (-jnp.exp(vp['A_logs'])).transpose(0, 2, 1).astype(F32),
            ]
            ysl, yst = _call(
                _ssm_kernel, "ssm", B, [xc, xct], ssm_w,
                [jax.ShapeDtypeStruct((B, L, Di), F32),
                 jax.ShapeDtypeStruct((B, L, Di), F32)],
                scratch=[pltpu.VMEM((2, L, Di), F32),
                         pltpu.VMEM((2, L, 16), F32),
                         pltpu.VMEM((2, L, 16), F32)])
            ystu = _tr_hw(yst, H, W)
            post_w = [
                _row(vp['out_norm_g']), _row(vp['out_norm_b']),
                _row(vp['Ds'].sum(axis=0)),
                vp['out_proj_w'].T.astype(F32),
                _row(vp['ln2_g']), _row(vp['ln2_b']),
                vp['mlp_w1'].T.astype(F32), _row(vp['mlp_b1']),
                vp['mlp_w2'].T.astype(F32), _row(vp['mlp_b2']),
            ]
            h = _call(_post_kernel, "vss_post", B, [ysl, ystu, xc, z, h],
                      post_w, [jax.ShapeDtypeStruct((B, L, d), F32)])
        rp = layer['res']
        res_w = [
            _row(rp['gn1_g']), _row(rp['gn1_b']), _cw(rp['conv1_w']),
            _row(rp['conv1_b']), _row(rp['gn2_g']), _row(rp['gn2_b']),
            _cw(rp['conv2_w']), _row(rp['conv2_b']), gmb, gmb.T,
        ]
        h = _call(_res_kernel, "res", B, [h], res_w,
                  [jax.ShapeDtypeStruct((B, L, d), F32)])

    return h.reshape(B, H, W, d).transpose(0, 3, 1, 2)
